# baseline (speedup 1.0000x reference)
import jax
import jax.numpy as jnp
from jax import lax
from jax.experimental import pallas as pl
from jax.experimental.pallas import tpu as pltpu

_B, _S, _D = 4, 8192, 4096
_E, _K = 64, 8
_BT = 512


def _gate_kernel(x_ref, w_ref, tw_ref, ti_ref):
    x = x_ref[...]
    w = w_ref[...]
    logits = lax.dot_general(
        x, w, (((1,), (1,)), ((), ())),
        preferred_element_type=jnp.float32)

    C = 64
    invf = (jnp.int32(63)
            - lax.broadcasted_iota(jnp.int32, (C, _E), 1)).astype(jnp.float32)
    kcol = lax.broadcasted_iota(jnp.int32, (C, _K), 1)
    for c in range(_BT // C):
        g = logits[c * C:(c + 1) * C, :]
        sel_m = jnp.zeros((C, _K), jnp.float32)
        sel_if = jnp.zeros((C, _K), jnp.float32)
        for k in range(_K):
            m = jnp.max(g, axis=-1, keepdims=True)
            cand = jnp.where(g == m, invf, -1.0)
            af = jnp.max(cand, axis=-1, keepdims=True)
            sel_m = jnp.where(kcol == k, m, sel_m)
            sel_if = jnp.where(kcol == k, af, sel_if)
            g = jnp.where(cand == af, -jnp.inf, g)

        idx_k = jnp.int32(63) - sel_if.astype(jnp.int32)
        wsel = jax.nn.sigmoid(sel_m)
        wts = wsel / jnp.sum(wsel, axis=-1, keepdims=True)
        tw_ref[c * C:(c + 1) * C, :] = wts
        ti_ref[c * C:(c + 1) * C, :] = idx_k


def kernel(x, W):
    T = _B * _S
    xf = x.reshape(T, _D)
    grid = (T // _BT,)
    tw, ti = pl.pallas_call(
        _gate_kernel,
        grid=grid,
        in_specs=[
            pl.BlockSpec((_BT, _D), lambda i: (i, 0)),
            pl.BlockSpec((_E, _D), lambda i: (0, 0)),
        ],
        out_specs=[
            pl.BlockSpec((_BT, _K), lambda i: (i, 0)),
            pl.BlockSpec((_BT, _K), lambda i: (i, 0)),
        ],
        out_shape=[
            jax.ShapeDtypeStruct((T, _K), jnp.float32),
            jax.ShapeDtypeStruct((T, _K), jnp.int32),
        ],
    )(xf, W)
    return tw.reshape(_B, _S, _K), ti.reshape(_B, _S, _K)


# speedup vs baseline: 1.1052x; 1.1052x over previous
import jax
import jax.numpy as jnp
from jax import lax
from jax.experimental import pallas as pl
from jax.experimental.pallas import tpu as pltpu

_B, _S, _D = 4, 8192, 4096
_E, _K = 64, 8
_BT = 512


def _gate_kernel(x_ref, w_ref, tw_ref, ti_ref):
    x = x_ref[...]
    w = w_ref[...]
    gt = lax.dot_general(
        w, x, (((1,), (1,)), ((), ())),
        preferred_element_type=jnp.float32)

    invf = (jnp.int32(63)
            - lax.broadcasted_iota(jnp.int32, (_E, _BT), 0)).astype(jnp.float32)
    krow = lax.broadcasted_iota(jnp.int32, (_K, _BT), 0)
    sel_m = jnp.zeros((_K, _BT), jnp.float32)
    sel_if = jnp.zeros((_K, _BT), jnp.float32)
    for k in range(_K):
        m = jnp.max(gt, axis=0, keepdims=True)
        cand = jnp.where(gt == m, invf, -1.0)
        af = jnp.max(cand, axis=0, keepdims=True)
        sel_m = jnp.where(krow == k, m, sel_m)
        sel_if = jnp.where(krow == k, af, sel_if)
        gt = jnp.where(cand == af, -jnp.inf, gt)

    idx_k = jnp.int32(63) - sel_if.astype(jnp.int32)
    wsel = jax.nn.sigmoid(sel_m)
    wts = wsel / jnp.sum(wsel, axis=0, keepdims=True)
    tw_ref[...] = wts.T
    ti_ref[...] = idx_k.T


def kernel(x, W):
    T = _B * _S
    xf = x.reshape(T, _D)
    grid = (T // _BT,)
    tw, ti = pl.pallas_call(
        _gate_kernel,
        grid=grid,
        in_specs=[
            pl.BlockSpec((_BT, _D), lambda i: (i, 0)),
            pl.BlockSpec((_E, _D), lambda i: (0, 0)),
        ],
        out_specs=[
            pl.BlockSpec((_BT, _K), lambda i: (i, 0)),
            pl.BlockSpec((_BT, _K), lambda i: (i, 0)),
        ],
        out_shape=[
            jax.ShapeDtypeStruct((T, _K), jnp.float32),
            jax.ShapeDtypeStruct((T, _K), jnp.int32),
        ],
    )(xf, W)
    return tw.reshape(_B, _S, _K), ti.reshape(_B, _S, _K)


# speedup vs baseline: 1.1076x; 1.0021x over previous
import jax
import jax.numpy as jnp
from jax import lax
from jax.experimental import pallas as pl
from jax.experimental.pallas import tpu as pltpu

_B, _S, _D = 4, 8192, 4096
_E, _K = 64, 8
_BT = 512


def _gate_kernel(x_ref, w_ref, tw_ref, ti_ref):
    x = x_ref[...]
    w = w_ref[...]
    gt = lax.dot_general(
        w, x, (((1,), (1,)), ((), ())),
        preferred_element_type=jnp.float32)

    SEG = 2
    sw = _BT // SEG
    invf = (jnp.int32(63)
            - lax.broadcasted_iota(jnp.int32, (_E, sw), 0)).astype(jnp.float32)
    krow = lax.broadcasted_iota(jnp.int32, (_K, sw), 0)
    gts = [gt[:, s * sw:(s + 1) * sw] for s in range(SEG)]
    sel_m = [jnp.zeros((_K, sw), jnp.float32) for _ in range(SEG)]
    sel_if = [jnp.zeros((_K, sw), jnp.float32) for _ in range(SEG)]
    for k in range(_K):
        for s in range(SEG):
            m = jnp.max(gts[s], axis=0, keepdims=True)
            cand = jnp.where(gts[s] == m, invf, -1.0)
            af = jnp.max(cand, axis=0, keepdims=True)
            sel_m[s] = jnp.where(krow == k, m, sel_m[s])
            sel_if[s] = jnp.where(krow == k, af, sel_if[s])
            gts[s] = jnp.where(cand == af, -jnp.inf, gts[s])

    for s in range(SEG):
        idx_k = jnp.int32(63) - sel_if[s].astype(jnp.int32)
        wsel = jax.nn.sigmoid(sel_m[s])
        wts = wsel / jnp.sum(wsel, axis=0, keepdims=True)
        tw_ref[s * sw:(s + 1) * sw, :] = wts.T
        ti_ref[s * sw:(s + 1) * sw, :] = idx_k.T


def kernel(x, W):
    T = _B * _S
    xf = x.reshape(T, _D)
    grid = (T // _BT,)
    tw, ti = pl.pallas_call(
        _gate_kernel,
        grid=grid,
        in_specs=[
            pl.BlockSpec((_BT, _D), lambda i: (i, 0)),
            pl.BlockSpec((_E, _D), lambda i: (0, 0)),
        ],
        out_specs=[
            pl.BlockSpec((_BT, _K), lambda i: (i, 0)),
            pl.BlockSpec((_BT, _K), lambda i: (i, 0)),
        ],
        out_shape=[
            jax.ShapeDtypeStruct((T, _K), jnp.float32),
            jax.ShapeDtypeStruct((T, _K), jnp.int32),
        ],
    )(xf, W)
    return tw.reshape(_B, _S, _K), ti.reshape(_B, _S, _K)


# speedup vs baseline: 1.1900x; 1.0744x over previous
import jax
import jax.numpy as jnp
from jax import lax
from jax.experimental import pallas as pl
from jax.experimental.pallas import tpu as pltpu

_B, _S, _D = 4, 8192, 4096
_E, _K = 64, 8
_BT = 1024


def _gate_kernel(x_ref, w_ref, tw_ref, ti_ref):
    x = x_ref[...]
    w = w_ref[...]
    gt = lax.dot_general(
        w, x, (((1,), (1,)), ((), ())),
        preferred_element_type=jnp.float32)

    SEG = 2
    sw = _BT // SEG
    invf = (jnp.int32(63)
            - lax.broadcasted_iota(jnp.int32, (_E, sw), 0)).astype(jnp.float32)
    krow = lax.broadcasted_iota(jnp.int32, (_K, sw), 0)
    gts = [gt[:, s * sw:(s + 1) * sw] for s in range(SEG)]
    sel_m = [jnp.zeros((_K, sw), jnp.float32) for _ in range(SEG)]
    sel_if = [jnp.zeros((_K, sw), jnp.float32) for _ in range(SEG)]
    for k in range(_K):
        for s in range(SEG):
            m = jnp.max(gts[s], axis=0, keepdims=True)
            cand = jnp.where(gts[s] == m, invf, -1.0)
            af = jnp.max(cand, axis=0, keepdims=True)
            sel_m[s] = jnp.where(krow == k, m, sel_m[s])
            sel_if[s] = jnp.where(krow == k, af, sel_if[s])
            gts[s] = jnp.where(cand == af, -jnp.inf, gts[s])

    for s in range(SEG):
        idx_k = jnp.int32(63) - sel_if[s].astype(jnp.int32)
        wsel = jax.nn.sigmoid(sel_m[s])
        wts = wsel / jnp.sum(wsel, axis=0, keepdims=True)
        tw_ref[s * sw:(s + 1) * sw, :] = wts.T
        ti_ref[s * sw:(s + 1) * sw, :] = idx_k.T


def kernel(x, W):
    T = _B * _S
    xf = x.reshape(T, _D)
    grid = (T // _BT,)
    tw, ti = pl.pallas_call(
        _gate_kernel,
        grid=grid,
        in_specs=[
            pl.BlockSpec((_BT, _D), lambda i: (i, 0)),
            pl.BlockSpec((_E, _D), lambda i: (0, 0)),
        ],
        out_specs=[
            pl.BlockSpec((_BT, _K), lambda i: (i, 0)),
            pl.BlockSpec((_BT, _K), lambda i: (i, 0)),
        ],
        out_shape=[
            jax.ShapeDtypeStruct((T, _K), jnp.float32),
            jax.ShapeDtypeStruct((T, _K), jnp.int32),
        ],
    )(xf, W)
    return tw.reshape(_B, _S, _K), ti.reshape(_B, _S, _K)


# speedup vs baseline: 1.1903x; 1.0003x over previous
import jax
import jax.numpy as jnp
from jax import lax
from jax.experimental import pallas as pl
from jax.experimental.pallas import tpu as pltpu

_B, _S, _D = 4, 8192, 4096
_E, _K = 64, 8
_BT = 1024


def _gate_kernel(x_ref, w_ref, tw_ref, ti_ref):
    x = x_ref[...]
    w = w_ref[...]
    gt = lax.dot_general(
        w, x, (((1,), (1,)), ((), ())),
        preferred_element_type=jnp.float32)

    SEG = 4
    sw = _BT // SEG
    invf = (jnp.int32(63)
            - lax.broadcasted_iota(jnp.int32, (_E, sw), 0)).astype(jnp.float32)
    krow = lax.broadcasted_iota(jnp.int32, (_K, sw), 0)
    gts = [gt[:, s * sw:(s + 1) * sw] for s in range(SEG)]
    sel_m = [jnp.zeros((_K, sw), jnp.float32) for _ in range(SEG)]
    sel_if = [jnp.zeros((_K, sw), jnp.float32) for _ in range(SEG)]
    for k in range(_K):
        for s in range(SEG):
            m = jnp.max(gts[s], axis=0, keepdims=True)
            cand = jnp.where(gts[s] == m, invf, -1.0)
            af = jnp.max(cand, axis=0, keepdims=True)
            sel_m[s] = jnp.where(krow == k, m, sel_m[s])
            sel_if[s] = jnp.where(krow == k, af, sel_if[s])
            gts[s] = jnp.where(cand == af, -jnp.inf, gts[s])

    for s in range(SEG):
        idx_k = jnp.int32(63) - sel_if[s].astype(jnp.int32)
        wsel = jax.nn.sigmoid(sel_m[s])
        wts = wsel / jnp.sum(wsel, axis=0, keepdims=True)
        tw_ref[s * sw:(s + 1) * sw, :] = wts.T
        ti_ref[s * sw:(s + 1) * sw, :] = idx_k.T


def kernel(x, W):
    T = _B * _S
    xf = x.reshape(T, _D)
    grid = (T // _BT,)
    tw, ti = pl.pallas_call(
        _gate_kernel,
        grid=grid,
        in_specs=[
            pl.BlockSpec((_BT, _D), lambda i: (i, 0)),
            pl.BlockSpec((_E, _D), lambda i: (0, 0)),
        ],
        out_specs=[
            pl.BlockSpec((_BT, _K), lambda i: (i, 0)),
            pl.BlockSpec((_BT, _K), lambda i: (i, 0)),
        ],
        out_shape=[
            jax.ShapeDtypeStruct((T, _K), jnp.float32),
            jax.ShapeDtypeStruct((T, _K), jnp.int32),
        ],
    )(xf, W)
    return tw.reshape(_B, _S, _K), ti.reshape(_B, _S, _K)


# speedup vs baseline: 1.2009x; 1.0089x over previous
import jax
import jax.numpy as jnp
from jax import lax
from jax.experimental import pallas as pl

_B, _S, _D = 4, 8192, 4096
_E, _K = 64, 8
_BT = 1024
_SEG = 4


def _gate_kernel(x_ref, w_ref, tw_ref, ti_ref):
    x = x_ref[...]
    w = w_ref[...]
    gt = lax.dot_general(
        w, x, (((1,), (1,)), ((), ())),
        preferred_element_type=jnp.float32)

    sw = _BT // _SEG
    invf = (jnp.int32(63)
            - lax.broadcasted_iota(jnp.int32, (_E, sw), 0)).astype(jnp.float32)
    krow = lax.broadcasted_iota(jnp.int32, (_K, sw), 0)
    gts = [gt[:, s * sw:(s + 1) * sw] for s in range(_SEG)]
    sel_m = [jnp.zeros((_K, sw), jnp.float32) for _ in range(_SEG)]
    sel_if = [jnp.zeros((_K, sw), jnp.float32) for _ in range(_SEG)]
    for k in range(_K):
        for s in range(_SEG):
            m = jnp.max(gts[s], axis=0, keepdims=True)
            cand = jnp.where(gts[s] == m, invf, -1.0)
            af = jnp.max(cand, axis=0, keepdims=True)
            sel_m[s] = jnp.where(krow == k, m, sel_m[s])
            sel_if[s] = jnp.where(krow == k, af, sel_if[s])
            gts[s] = jnp.where(cand == af, -jnp.inf, gts[s])

    for s in range(_SEG):
        idx_k = jnp.int32(63) - sel_if[s].astype(jnp.int32)
        wsel = jax.nn.sigmoid(sel_m[s])
        wts = wsel / jnp.sum(wsel, axis=0, keepdims=True)
        tw_ref[s * sw:(s + 1) * sw, :] = wts.T
        ti_ref[s * sw:(s + 1) * sw, :] = idx_k.T


def kernel(x, W):
    T = _B * _S
    xf = x.reshape(T, _D)
    grid = (T // _BT,)
    tw, ti = pl.pallas_call(
        _gate_kernel,
        grid=grid,
        in_specs=[
            pl.BlockSpec((_BT, _D), lambda i: (i, 0)),
            pl.BlockSpec((_E, _D), lambda i: (0, 0)),
        ],
        out_specs=[
            pl.BlockSpec((_BT, _K), lambda i: (i, 0)),
            pl.BlockSpec((_BT, _K), lambda i: (i, 0)),
        ],
        out_shape=[
            jax.ShapeDtypeStruct((T, _K), jnp.float32),
            jax.ShapeDtypeStruct((T, _K), jnp.int32),
        ],
    )(xf, W)
    return tw.reshape(_B, _S, _K), ti.reshape(_B, _S, _K)
